# baseline (speedup 1.0000x reference)
import jax
import jax.numpy as jnp
from jax.experimental import pallas as pl

_BM = 400


def _support_kernel(feature_ref, w_ref, out_ref, out_bf16_ref):
    acc = jnp.dot(feature_ref[...], w_ref[...], preferred_element_type=jnp.float32)
    sup = jnp.maximum(acc, 0.0)
    out_ref[...] = sup
    out_bf16_ref[...] = sup.astype(jnp.bfloat16)


def _agg_kernel(adj_ref, support_ref, support_diag_ref, b_ref, out_ref):
    acc = jnp.dot(
        adj_ref[...].astype(jnp.bfloat16),
        support_ref[...],
        preferred_element_type=jnp.float32,
    )
    out_ref[...] = acc + support_diag_ref[...] + b_ref[...]


def kernel(feature, adj, W, b):
    n, _ = feature.shape
    d = W.shape[1]

    support, support_bf16 = pl.pallas_call(
        _support_kernel,
        out_shape=(
            jax.ShapeDtypeStruct((n, d), jnp.float32),
            jax.ShapeDtypeStruct((n, d), jnp.bfloat16),
        ),
    )(feature, W)

    bm = _BM
    out = pl.pallas_call(
        _agg_kernel,
        grid=(n // bm,),
        in_specs=[
            pl.BlockSpec((bm, n), lambda i: (i, 0)),
            pl.BlockSpec((n, d), lambda i: (0, 0)),
            pl.BlockSpec((bm, d), lambda i: (i, 0)),
            pl.BlockSpec((1, d), lambda i: (0, 0)),
        ],
        out_specs=pl.BlockSpec((bm, d), lambda i: (i, 0)),
        out_shape=jax.ShapeDtypeStruct((n, d), jnp.float32),
    )(adj, support_bf16, support, b.reshape(1, d))
    return out


# speedup vs baseline: 1.0030x; 1.0030x over previous
import jax
import jax.numpy as jnp
from jax.experimental import pallas as pl

_BM = 200
_SPLIT = 2


def _support_kernel(feature_ref, w_ref, out_ref, out_bf16_ref):
    acc = jnp.dot(feature_ref[...], w_ref[...], preferred_element_type=jnp.float32)
    sup = jnp.maximum(acc, 0.0)
    out_ref[...] = sup
    out_bf16_ref[...] = sup.astype(jnp.bfloat16)


def _agg_kernel(*refs):
    s = _SPLIT
    adj_refs = refs[:s]
    support_ref = refs[s]
    diag_refs = refs[s + 1:2 * s + 1]
    b_ref = refs[2 * s + 1]
    out_refs = refs[2 * s + 2:]
    sup = support_ref[...]
    bias = b_ref[...]
    for a_ref, d_ref, o_ref in zip(adj_refs, diag_refs, out_refs):
        acc = jnp.dot(
            a_ref[...].astype(jnp.bfloat16), sup,
            preferred_element_type=jnp.float32,
        )
        o_ref[...] = acc + d_ref[...] + bias


def kernel(feature, adj, W, b):
    n, _ = feature.shape
    d = W.shape[1]

    support, support_bf16 = pl.pallas_call(
        _support_kernel,
        out_shape=(
            jax.ShapeDtypeStruct((n, d), jnp.float32),
            jax.ShapeDtypeStruct((n, d), jnp.bfloat16),
        ),
    )(feature, W)

    bm, s = _BM, _SPLIT
    seg = n // s
    steps = seg // bm

    adj_specs = [
        pl.BlockSpec((bm, n), lambda i, s_=s_: (s_ * steps + i, 0))
        for s_ in range(s)
    ]
    diag_specs = [
        pl.BlockSpec((bm, d), lambda i, s_=s_: (s_ * steps + i, 0))
        for s_ in range(s)
    ]
    outs = pl.pallas_call(
        _agg_kernel,
        grid=(steps,),
        in_specs=adj_specs
        + [pl.BlockSpec((n, d), lambda i: (0, 0))]
        + diag_specs
        + [pl.BlockSpec((1, d), lambda i: (0, 0))],
        out_specs=[pl.BlockSpec((bm, d), lambda i: (i, 0))] * s,
        out_shape=[jax.ShapeDtypeStruct((seg, d), jnp.float32)] * s,
    )(*([adj] * s), support_bf16, *([support] * s), b.reshape(1, d))
    return jnp.concatenate(outs, axis=0)
